# Initial kernel scaffold; baseline (speedup 1.0000x reference)
import jax
import jax.numpy as jnp
from jax.experimental import pallas as pl


def kernel(x, adj, W0, W1):
    raise NotImplementedError("write your pallas kernel here")


# speedup vs baseline: 1.0174x; 1.0174x over previous
import functools

import jax
import jax.numpy as jnp
import numpy as np
from jax.experimental import pallas as pl
from jax.experimental.pallas import tpu as pltpu

N = 4096
C = 128
SNR_LIN = 10.0
SQRT_HALF = float(np.sqrt(0.5))

KR = ((0xE14166EC, 0x9EC84F81), (0x04658493, 0x009F6A70))
KI = ((0x61F15A13, 0x246FE96D), (0x65D0DF45, 0xA542AECB))
KN = ((0xA1495F6E, 0x9D577F1C), (0x840A05C2, 0x088E666D))

BM = 256
BK = 512


def _threefry_bits(k1, k2, counts_lo):
    u32 = jnp.uint32
    ks0 = u32(k1)
    ks1 = u32(k2)
    ks2 = u32(k1 ^ k2 ^ 0x1BD11BDA)

    x0 = jnp.full_like(counts_lo, ks0)
    x1 = counts_lo + ks1

    def rotl(v, r):
        return (v << u32(r)) | jax.lax.shift_right_logical(v, u32(32 - r))

    def four_rounds(x0, x1, rots):
        for r in rots:
            x0 = x0 + x1
            x1 = rotl(x1, r)
            x1 = x0 ^ x1
        return x0, x1

    ra = (13, 15, 26, 6)
    rb = (17, 29, 16, 24)
    x0, x1 = four_rounds(x0, x1, ra)
    x0, x1 = x0 + ks1, x1 + ks2 + u32(1)
    x0, x1 = four_rounds(x0, x1, rb)
    x0, x1 = x0 + ks2, x1 + ks0 + u32(2)
    x0, x1 = four_rounds(x0, x1, ra)
    x0, x1 = x0 + ks0, x1 + ks1 + u32(3)
    x0, x1 = four_rounds(x0, x1, rb)
    x0, x1 = x0 + ks1, x1 + ks2 + u32(4)
    x0, x1 = four_rounds(x0, x1, ra)
    x0, x1 = x0 + ks2, x1 + ks0 + u32(5)
    return x0 ^ x1


def _erf_inv(x):
    w = -jnp.log1p(-x * x)
    wa = w - 2.5
    pa = jnp.float32(2.81022636e-08)
    for c in (3.43273939e-07, -3.5233877e-06, -4.39150654e-06, 0.00021858087,
              -0.00125372503, -0.00417768164, 0.246640727, 1.50140941):
        pa = jnp.float32(c) + pa * wa
    wb = jnp.sqrt(w) - 3.0
    pb = jnp.float32(-0.000200214257)
    for c in (0.000100950558, 0.00134934322, -0.00367342844, 0.00573950773,
              -0.0076224613, 0.00943887047, 1.00167406, 2.83297682):
        pb = jnp.float32(c) + pb * wb
    p = jnp.where(w < 5.0, pa, pb)
    return p * x


def _normal_from_counts(key, counts_lo):
    bits = _threefry_bits(key[0], key[1], counts_lo)
    fbits = jax.lax.shift_right_logical(bits, jnp.uint32(9)) | jnp.uint32(0x3F800000)
    f = jax.lax.bitcast_convert_type(fbits, jnp.float32) - 1.0
    lo = jnp.float32(-0.9999999403953552)
    hi = jnp.float32(1.0)
    u = jnp.maximum(lo, f * (hi - lo) + lo)
    return jnp.float32(1.4142135381698608) * _erf_inv(u)


def _shift_matmul_kernel(adj_ref, x_ref, y_ref, acc_ref, *, kr, ki, n_k):
    i = pl.program_id(0)
    k = pl.program_id(1)
    rows = jax.lax.broadcasted_iota(jnp.uint32, (BM, BK), 0)
    cols = jax.lax.broadcasted_iota(jnp.uint32, (BM, BK), 1)
    base = (i.astype(jnp.uint32) * jnp.uint32(BM)) * jnp.uint32(N) + \
        k.astype(jnp.uint32) * jnp.uint32(BK)
    flat = base + rows * jnp.uint32(N) + cols
    re = _normal_from_counts(kr, flat) * SQRT_HALF
    im = _normal_from_counts(ki, flat) * SQRT_HALF
    fad = jnp.sqrt(re * re + im * im) * SQRT_HALF
    s = adj_ref[...] * fad
    contrib = jax.lax.dot_general(
        s, x_ref[...], (((1,), (0,)), ((), ())),
        preferred_element_type=jnp.float32,
        precision=jax.lax.Precision.HIGHEST)

    @pl.when(k == 0)
    def _():
        acc_ref[...] = contrib

    @pl.when(k != 0)
    def _():
        acc_ref[...] += contrib

    @pl.when(k == n_k - 1)
    def _():
        y_ref[...] = acc_ref[...]


def _shift_matmul(adj, x2d, kr, ki):
    n_i, n_k = N // BM, N // BK
    body = functools.partial(_shift_matmul_kernel, kr=kr, ki=ki, n_k=n_k)
    return pl.pallas_call(
        body,
        grid=(n_i, n_k),
        in_specs=[
            pl.BlockSpec((BM, BK), lambda i, k: (i, k)),
            pl.BlockSpec((BK, C), lambda i, k: (k, 0)),
        ],
        out_specs=pl.BlockSpec((BM, C), lambda i, k: (i, 0)),
        out_shape=jax.ShapeDtypeStruct((N, C), jnp.float32),
        scratch_shapes=[pltpu.VMEM((BM, C), jnp.float32)],
        compiler_params=pltpu.CompilerParams(
            dimension_semantics=("parallel", "arbitrary")),
    )(adj, x2d)


def _combine_kernel(y_ref, w_ref, prev_ref, x_out_ref, out_ref, *, kn):
    y = y_ref[...]
    x_power = jnp.sum(y * y) / jnp.float32(N * C)
    std = jnp.sqrt(x_power / jnp.float32(SNR_LIN))
    rows = jax.lax.broadcasted_iota(jnp.uint32, (N, C), 0)
    cols = jax.lax.broadcasted_iota(jnp.uint32, (N, C), 1)
    noise = _normal_from_counts(kn, rows * jnp.uint32(C) + cols)
    x_new = y + noise * std
    x_out_ref[...] = x_new
    contrib = jax.lax.dot_general(
        x_new, w_ref[...], (((1,), (1,)), ((), ())),
        preferred_element_type=jnp.float32,
        precision=jax.lax.Precision.HIGHEST)
    out_ref[...] = prev_ref[...] + contrib


def _combine(y, w, prev, kn):
    body = functools.partial(_combine_kernel, kn=kn)
    return pl.pallas_call(
        body,
        in_specs=[pl.BlockSpec(memory_space=pltpu.VMEM)] * 3,
        out_specs=[pl.BlockSpec(memory_space=pltpu.VMEM)] * 2,
        out_shape=[
            jax.ShapeDtypeStruct((N, C), jnp.float32),
            jax.ShapeDtypeStruct((N, C), jnp.float32),
        ],
    )(y, w, prev)


def kernel(x, adj, W0, W1):
    x2d = x[0]
    y0 = _shift_matmul(adj, x2d, KR[0], KI[0])
    x1, out0 = _combine(y0, W0, jnp.zeros((N, C), jnp.float32), KN[0])
    y1 = _shift_matmul(adj, x1, KR[1], KI[1])
    _, out = _combine(y1, W1, out0, KN[1])
    return out[None]
